# baseline (speedup 1.0000x reference)
import jax
import jax.numpy as jnp
from jax.experimental import pallas as pl


def _copy_body(x_ref, o_ref):
    o_ref[...] = x_ref[...]


@jax.jit
def kernel(features, img_path):
    del img_path
    B, NT, D = features.shape
    flat = features.reshape(B * NT, D)
    out = pl.pallas_call(
        _copy_body,
        grid=(128,),
        in_specs=[pl.BlockSpec((512, D), lambda j: (j, 0))],
        out_specs=pl.BlockSpec((512, D), lambda j: (j, 0)),
        out_shape=jax.ShapeDtypeStruct((128 * 512, D), jnp.float32),
    )(flat)
    return out[: B * (NT - 1)].reshape(B, NT - 1, D)


# speedup vs baseline: 1.7426x; 1.7426x over previous
import functools

import jax
import jax.numpy as jnp
from jax.experimental import pallas as pl

_RATIO = 0.5


def _select_body(x_ref, o_ref, *, k):
    x = x_ref[0]
    n_tok = x.shape[0]
    n = n_tok - 1

    scores = jnp.sum(x, axis=1, keepdims=True) * (1.0 / x.shape[1])

    bits = jax.lax.bitcast_convert_type(scores, jnp.int32)
    key = bits ^ ((bits >> 31) & jnp.int32(0x7FFFFFFF))
    int_min = jnp.int32(-2147483648)
    tok_idx = jax.lax.broadcasted_iota(jnp.int32, key.shape, 0)
    key = jnp.where(tok_idx == 0, int_min, key)
    u = key ^ int_min

    kk = jnp.int32(k)
    vals = jax.lax.broadcasted_iota(jnp.int32, (1, 256), 1)

    match = jnp.ones(key.shape, dtype=jnp.bool_)
    lo_cnt = jnp.int32(0)
    thr_off = jnp.int32(0)
    for shift in (24, 16, 8, 0):
        byte = jax.lax.shift_right_logical(u, shift) & jnp.int32(0xFF)
        bytes_m = jnp.where(match, byte, jnp.int32(-1))
        need = kk - lo_cnt
        s_v = jnp.sum((bytes_m >= vals).astype(jnp.int32), axis=0,
                      keepdims=True)
        vstar = jnp.sum((s_v >= need).astype(jnp.int32)) - 1
        lo_cnt = lo_cnt + jnp.sum((bytes_m > vstar).astype(jnp.int32))
        match = match & (byte == vstar)
        thr_off = thr_off | (vstar << shift)

    thr = thr_off ^ int_min
    gt = key > thr
    need = kk - lo_cnt
    n_ties = jnp.sum(match.astype(jnp.int32))

    def exact_ties(_):
        def idx_step(_, lohi):
            lo, hi = lohi
            mid = (lo + hi) // 2
            cnt = jnp.sum((match & (tok_idx < mid)).astype(jnp.int32))
            ok = cnt >= need
            return jnp.where(ok, lo, mid), jnp.where(ok, mid, hi)

        _, bound = jax.lax.fori_loop(
            0, 12, idx_step, (jnp.int32(0), jnp.int32(n_tok)))
        return bound

    bound = jax.lax.cond(n_ties == need, lambda _: jnp.int32(n_tok),
                         exact_ties, operand=None)
    mask = gt | (match & (tok_idx < bound))
    o_ref[0] = x[1:] * mask[1:].astype(jnp.float32)


@jax.jit
def kernel(features, img_path):
    del img_path
    B, NT, D = features.shape
    N = NT - 1
    k = int(N * _RATIO)
    body = functools.partial(_select_body, k=k)
    return pl.pallas_call(
        body,
        grid=(B,),
        in_specs=[pl.BlockSpec((1, NT, D), lambda b: (b, 0, 0))],
        out_specs=pl.BlockSpec((1, N, D), lambda b: (b, 0, 0)),
        out_shape=jax.ShapeDtypeStruct((B, N, D), jnp.float32),
    )(features)
